# baseline (speedup 1.0000x reference)
import functools
import math

import jax
import jax.numpy as jnp
import numpy as np
from jax import lax
from jax.experimental import pallas as pl
from jax.experimental.pallas import tpu as pltpu
from jax.experimental.pallas import tpu_sc as plsc


def _sc_gather(memory, idx):
    A_, S_, D_ = memory.shape
    B_ = idx.shape[0]
    NC, NS = 2, 16
    NW = NC * NS
    bpw = B_ // NW
    G = 8
    nit = bpw // G

    mesh = plsc.VectorSubcoreMesh(core_axis_name="c", subcore_axis_name="s")

    @functools.partial(
        pl.kernel,
        mesh=mesh,
        out_type=jax.ShapeDtypeStruct((S_, B_, D_), jnp.float32),
        scratch_types=[
            pltpu.VMEM((bpw,), jnp.int32),
            pltpu.VMEM((2, G, S_, D_), jnp.float32),
            pltpu.SemaphoreType.DMA,
            pltpu.SemaphoreType.DMA,
        ],
    )
    def gather_kernel(mem_hbm, idx_hbm, out_hbm, idx_v, rows_v, sem_g, sem_w):
        wid = lax.axis_index("s") * NC + lax.axis_index("c")
        base = wid * bpw
        pltpu.sync_copy(idx_hbm.at[pl.ds(base, bpw)], idx_v)

        def gstart(i, slot):
            off = pl.multiple_of(i * G, 8)
            pltpu.async_copy(
                mem_hbm.at[idx_v.at[pl.ds(off, G)]], rows_v.at[slot], sem_g)

        def gwait(slot):
            pltpu.make_async_copy(
                mem_hbm.at[idx_v.at[pl.ds(0, G)]], rows_v.at[slot],
                sem_g).wait()

        def wstart(i, slot):
            off = pl.multiple_of(base + i * G, 8)
            for s in range(S_):
                pltpu.async_copy(
                    rows_v.at[slot, :, s, :],
                    out_hbm.at[s, pl.ds(off, G)], sem_w)

        def wwait(slot):
            for s in range(S_):
                pltpu.make_async_copy(
                    rows_v.at[slot, :, s, :],
                    out_hbm.at[s, pl.ds(0, G)], sem_w).wait()

        gstart(0, 0)

        def body(i2, carry):
            for j in range(2):
                i = i2 * 2 + j
                slot = j
                nslot = 1 - j

                @pl.when(i > 0)
                def _():
                    wwait(nslot)

                @pl.when(i + 1 < nit)
                def _():
                    gstart(i + 1, nslot)

                gwait(slot)
                wstart(i, slot)
            return carry

        lax.fori_loop(0, nit // 2, body, 0)
        wwait((nit - 1) % 2)

    return gather_kernel(memory, idx)


def _tc_attend(queries, gathered, e0f, selb, ktb, Wq, bq, Wf2b, bf,
               ln_g, ln_b, S_, D_, H_, interpret=False):
    B_, Q_ = queries.shape
    BBLK = 512
    grid = (B_ // BBLK,)
    scale = 1.0 / math.sqrt(D_)
    f32 = jnp.float32

    def body(q_ref, g_ref, e0_ref, k_ref, kt_ref, wq_ref, bq_ref, wf2_ref,
             bf_ref, gln_ref, bln_ref, o_ref):
        bf16 = jnp.bfloat16
        q = q_ref[...]
        pq = jnp.dot(q, wq_ref[...],
                     preferred_element_type=f32) + bq_ref[...]
        e0 = e0_ref[...]
        qcat = jnp.concatenate(
            [pq[:, h * D_:(h + 1) * D_] for h in range(H_)], axis=0)
        sq4 = jnp.dot(qcat * qcat, e0,
                      preferred_element_type=f32)[:, 0:1]
        qn4 = (qcat / jnp.maximum(jnp.sqrt(sq4), 1e-12)).astype(bf16)

        msb = [g_ref[s].astype(bf16) for s in range(S_)]
        m_cat = jnp.concatenate(msb, axis=1)
        ssq = jnp.dot(m_cat * m_cat, k_ref[...],
                      preferred_element_type=f32)
        inv_m = 1.0 / jnp.maximum(jnp.sqrt(ssq), 1e-12)

        ph = jnp.concatenate(
            [jnp.concatenate(
                [qn4[h * BBLK:(h + 1) * BBLK, :] * m for m in msb], axis=1)
             for h in range(H_)], axis=0)
        t4 = jnp.dot(ph, k_ref[...], preferred_element_type=f32)
        ws = []
        for h in range(H_):
            sc = t4[h * BBLK:(h + 1) * BBLK, :] * inv_m * scale
            e = jnp.exp(sc)
            ws.append(e / jnp.sum(e, axis=1, keepdims=True))
        w4 = jnp.concatenate(ws, axis=0).astype(bf16)
        wrep = jnp.dot(w4, kt_ref[...],
                       preferred_element_type=f32).astype(bf16)
        prod = jnp.concatenate(
            [wrep[h * BBLK:(h + 1) * BBLK, :] * m_cat for h in range(H_)],
            axis=1)
        hdn = jnp.dot(prod, wf2_ref[...],
                      preferred_element_type=f32) + bf_ref[...]
        hdn = 0.5 * hdn * (1.0 + lax.erf(hdn * (1.0 / math.sqrt(2.0))))
        mu = jnp.mean(hdn, axis=-1, keepdims=True)
        var = jnp.mean((hdn - mu) * (hdn - mu), axis=-1, keepdims=True)
        o_ref[...] = ((hdn - mu) / jnp.sqrt(var + 1e-5)) * gln_ref[...] \
            + bln_ref[...]

    return pl.pallas_call(
        body,
        grid=grid,
        in_specs=[
            pl.BlockSpec((BBLK, Q_), lambda i: (i, 0)),
            pl.BlockSpec((S_, BBLK, D_), lambda i: (0, i, 0)),
            pl.BlockSpec((D_, S_), lambda i: (0, 0)),
            pl.BlockSpec((S_ * D_, S_), lambda i: (0, 0)),
            pl.BlockSpec((S_, S_ * D_), lambda i: (0, 0)),
            pl.BlockSpec((Q_, H_ * D_), lambda i: (0, 0)),
            pl.BlockSpec((1, H_ * D_), lambda i: (0, 0)),
            pl.BlockSpec((H_ * S_ * D_, Q_), lambda i: (0, 0)),
            pl.BlockSpec((1, Q_), lambda i: (0, 0)),
            pl.BlockSpec((1, Q_), lambda i: (0, 0)),
            pl.BlockSpec((1, Q_), lambda i: (0, 0)),
        ],
        out_specs=pl.BlockSpec((BBLK, Q_), lambda i: (i, 0)),
        out_shape=jax.ShapeDtypeStruct((B_, Q_), jnp.float32),
        interpret=interpret,
    )(queries, gathered, e0f, selb, ktb, Wq, bq.reshape(1, -1), Wf2b,
      bf.reshape(1, -1), ln_g.reshape(1, -1), ln_b.reshape(1, -1))


def _selector(S_, D_):
    return (np.arange(S_ * D_, dtype=np.int32)[:, None] // D_
            == np.arange(S_, dtype=np.int32)[None, :]).astype(np.float32)


def kernel(queries, agent_indices, memory, Wq, bq, Wf, bf, ln_g, ln_b):
    A_, S_, D_ = memory.shape
    B_, Q_ = queries.shape
    H_ = Wq.shape[1] // D_
    idx = agent_indices.astype(jnp.int32)
    sel = _selector(S_, D_)
    e0f = jnp.asarray(sel[:D_, :])
    selb = jnp.asarray(sel, dtype=jnp.bfloat16)
    ktb = jnp.asarray(sel.T.copy(), dtype=jnp.bfloat16)
    Wf2b = jnp.concatenate(
        [jnp.tile(Wf[h * D_:(h + 1) * D_, :], (S_, 1)) for h in range(H_)],
        axis=0).astype(jnp.bfloat16)

    NCHUNK = 8
    Bc = B_ // NCHUNK
    outs = []
    for k in range(NCHUNK):
        idx_k = lax.dynamic_slice_in_dim(idx, k * Bc, Bc)
        q_k = lax.dynamic_slice_in_dim(queries, k * Bc, Bc)
        g_k = _sc_gather(memory, idx_k)
        outs.append(_tc_attend(q_k, g_k, e0f, selb, ktb, Wq, bq, Wf2b, bf,
                               ln_g, ln_b, S_, D_, H_))
    return jnp.concatenate(outs, axis=0)


# speedup vs baseline: 1.1085x; 1.1085x over previous
import functools
import math

import jax
import jax.numpy as jnp
import numpy as np
from jax import lax
from jax.experimental import pallas as pl
from jax.experimental.pallas import tpu as pltpu
from jax.experimental.pallas import tpu_sc as plsc


def _sc_gather(memory, idx):
    A_, S_, D_ = memory.shape
    B_ = idx.shape[0]
    NC, NS = 2, 16
    NW = NC * NS
    bpw = B_ // NW
    G = 8
    nit = bpw // G

    mesh = plsc.VectorSubcoreMesh(core_axis_name="c", subcore_axis_name="s")

    @functools.partial(
        pl.kernel,
        mesh=mesh,
        out_type=jax.ShapeDtypeStruct((S_, B_, D_), jnp.float32),
        scratch_types=[
            pltpu.VMEM((bpw,), jnp.int32),
            pltpu.VMEM((2, G, S_, D_), jnp.float32),
            pltpu.SemaphoreType.DMA,
            pltpu.SemaphoreType.DMA,
        ],
    )
    def gather_kernel(mem_hbm, idx_hbm, out_hbm, idx_v, rows_v, sem_g, sem_w):
        wid = lax.axis_index("s") * NC + lax.axis_index("c")
        base = wid * bpw
        pltpu.sync_copy(idx_hbm.at[pl.ds(base, bpw)], idx_v)

        def gstart(i, slot):
            off = pl.multiple_of(i * G, 8)
            pltpu.async_copy(
                mem_hbm.at[idx_v.at[pl.ds(off, G)]], rows_v.at[slot], sem_g)

        def gwait(slot):
            pltpu.make_async_copy(
                mem_hbm.at[idx_v.at[pl.ds(0, G)]], rows_v.at[slot],
                sem_g).wait()

        def wstart(i, slot):
            off = pl.multiple_of(base + i * G, 8)
            for s in range(S_):
                pltpu.async_copy(
                    rows_v.at[slot, :, s, :],
                    out_hbm.at[s, pl.ds(off, G)], sem_w)

        def wwait(slot):
            for s in range(S_):
                pltpu.make_async_copy(
                    rows_v.at[slot, :, s, :],
                    out_hbm.at[s, pl.ds(0, G)], sem_w).wait()

        gstart(0, 0)

        def body(i2, carry):
            for j in range(2):
                i = i2 * 2 + j
                slot = j
                nslot = 1 - j

                @pl.when(i > 0)
                def _():
                    wwait(nslot)

                @pl.when(i + 1 < nit)
                def _():
                    gstart(i + 1, nslot)

                gwait(slot)
                wstart(i, slot)
            return carry

        lax.fori_loop(0, nit // 2, body, 0)
        wwait((nit - 1) % 2)

    return gather_kernel(memory, idx)


def _tc_attend(queries, gathered, e0f, selb, ktb, Wq, bq, Wf2b, bf,
               ln_g, ln_b, S_, D_, H_, interpret=False):
    B_, Q_ = queries.shape
    BBLK = 512
    grid = (B_ // BBLK,)
    scale = 1.0 / math.sqrt(D_)
    f32 = jnp.float32

    def body(q_ref, g_ref, e0_ref, k_ref, kt_ref, wq_ref, bq_ref, wf2_ref,
             bf_ref, gln_ref, bln_ref, o_ref):
        bf16 = jnp.bfloat16
        q = q_ref[...]
        pq = jnp.dot(q, wq_ref[...],
                     preferred_element_type=f32) + bq_ref[...]
        e0 = e0_ref[...]
        qcat = jnp.concatenate(
            [pq[:, h * D_:(h + 1) * D_] for h in range(H_)], axis=0)
        sq4 = jnp.dot(qcat * qcat, e0,
                      preferred_element_type=f32)[:, 0:1]
        qn4 = (qcat / jnp.maximum(jnp.sqrt(sq4), 1e-12)).astype(bf16)

        msb = [g_ref[s].astype(bf16) for s in range(S_)]
        m_cat = jnp.concatenate(msb, axis=1)
        ssq = jnp.dot(m_cat * m_cat, k_ref[...],
                      preferred_element_type=f32)
        inv_m = 1.0 / jnp.maximum(jnp.sqrt(ssq), 1e-12)

        ph = jnp.concatenate(
            [jnp.concatenate(
                [qn4[h * BBLK:(h + 1) * BBLK, :] * m for m in msb], axis=1)
             for h in range(H_)], axis=0)
        t4 = jnp.dot(ph, k_ref[...], preferred_element_type=f32)
        ws = []
        for h in range(H_):
            sc = t4[h * BBLK:(h + 1) * BBLK, :] * inv_m * scale
            e = jnp.exp(sc)
            ws.append(e / jnp.sum(e, axis=1, keepdims=True))
        w4 = jnp.concatenate(ws, axis=0).astype(bf16)
        wrep = jnp.dot(w4, kt_ref[...],
                       preferred_element_type=f32).astype(bf16)
        prod = jnp.concatenate(
            [wrep[h * BBLK:(h + 1) * BBLK, :] * m_cat for h in range(H_)],
            axis=1)
        hdn = jnp.dot(prod, wf2_ref[...],
                      preferred_element_type=f32) + bf_ref[...]
        hdn = 0.5 * hdn * (1.0 + lax.erf(hdn * (1.0 / math.sqrt(2.0))))
        mu = jnp.mean(hdn, axis=-1, keepdims=True)
        var = jnp.mean((hdn - mu) * (hdn - mu), axis=-1, keepdims=True)
        o_ref[...] = ((hdn - mu) / jnp.sqrt(var + 1e-5)) * gln_ref[...] \
            + bln_ref[...]

    return pl.pallas_call(
        body,
        grid=grid,
        in_specs=[
            pl.BlockSpec((BBLK, Q_), lambda i: (i, 0)),
            pl.BlockSpec((S_, BBLK, D_), lambda i: (0, i, 0)),
            pl.BlockSpec((D_, S_), lambda i: (0, 0)),
            pl.BlockSpec((S_ * D_, S_), lambda i: (0, 0)),
            pl.BlockSpec((S_, S_ * D_), lambda i: (0, 0)),
            pl.BlockSpec((Q_, H_ * D_), lambda i: (0, 0)),
            pl.BlockSpec((1, H_ * D_), lambda i: (0, 0)),
            pl.BlockSpec((H_ * S_ * D_, Q_), lambda i: (0, 0)),
            pl.BlockSpec((1, Q_), lambda i: (0, 0)),
            pl.BlockSpec((1, Q_), lambda i: (0, 0)),
            pl.BlockSpec((1, Q_), lambda i: (0, 0)),
        ],
        out_specs=pl.BlockSpec((BBLK, Q_), lambda i: (i, 0)),
        out_shape=jax.ShapeDtypeStruct((B_, Q_), jnp.float32),
        interpret=interpret,
    )(queries, gathered, e0f, selb, ktb, Wq, bq.reshape(1, -1), Wf2b,
      bf.reshape(1, -1), ln_g.reshape(1, -1), ln_b.reshape(1, -1))


def _selector(S_, D_):
    return (np.arange(S_ * D_, dtype=np.int32)[:, None] // D_
            == np.arange(S_, dtype=np.int32)[None, :]).astype(np.float32)


def kernel(queries, agent_indices, memory, Wq, bq, Wf, bf, ln_g, ln_b):
    A_, S_, D_ = memory.shape
    B_, Q_ = queries.shape
    H_ = Wq.shape[1] // D_
    idx = agent_indices.astype(jnp.int32)
    sel = _selector(S_, D_)
    e0f = jnp.asarray(sel[:D_, :])
    selb = jnp.asarray(sel, dtype=jnp.bfloat16)
    ktb = jnp.asarray(sel.T.copy(), dtype=jnp.bfloat16)
    Wf2b = jnp.concatenate(
        [jnp.tile(Wf[h * D_:(h + 1) * D_, :], (S_, 1)) for h in range(H_)],
        axis=0).astype(jnp.bfloat16)

    NCHUNK = 4
    Bc = B_ // NCHUNK
    outs = []
    for k in range(NCHUNK):
        idx_k = lax.dynamic_slice_in_dim(idx, k * Bc, Bc)
        q_k = lax.dynamic_slice_in_dim(queries, k * Bc, Bc)
        g_k = _sc_gather(memory, idx_k)
        outs.append(_tc_attend(q_k, g_k, e0f, selb, ktb, Wq, bq, Wf2b, bf,
                               ln_g, ln_b, S_, D_, H_))
    return jnp.concatenate(outs, axis=0)
